# baseline (speedup 1.0000x reference)
import jax
import jax.numpy as jnp
from jax import lax
from jax.experimental import pallas as pl
from jax.experimental.pallas import tpu as pltpu
from jax.experimental.pallas import tpu_sc as plsc
import functools

BATCH = 16384
DIM = 64
NC = 2
NS = 16
NW = NC * NS
BPW = BATCH // NW
ICH = 128
NCH = BPW // ICH
NL = 16
PSTR = NL + 1


def _sc_body(uidx_hbm, iidx_hbm, emb_hbm, bias_hbm,
             out_hbm, uidx_v, iidx_v, urows_v, irows_v, ub_v, ib_v, pt_v,
             out_v, sem):
    wid = lax.axis_index("s") * NC + lax.axis_index("c")
    base = wid * BPW

    pltpu.sync_copy(uidx_hbm.at[wid], uidx_v)
    pltpu.sync_copy(iidx_hbm.at[wid], iidx_v)

    for j in range(NCH):
        pltpu.async_copy(emb_hbm.at[uidx_v.at[j]], urows_v.at[j], sem)
        pltpu.async_copy(emb_hbm.at[iidx_v.at[j]], irows_v.at[j], sem)
        pltpu.async_copy(bias_hbm.at[uidx_v.at[j]], ub_v.at[j], sem)
        pltpu.async_copy(bias_hbm.at[iidx_v.at[j]], ib_v.at[j], sem)
    for j in range(NCH):
        pltpu.make_async_copy(emb_hbm.at[uidx_v.at[j]], urows_v.at[j],
                              sem).wait()
        pltpu.make_async_copy(emb_hbm.at[iidx_v.at[j]], irows_v.at[j],
                              sem).wait()
        pltpu.make_async_copy(bias_hbm.at[uidx_v.at[j]], ub_v.at[j],
                              sem).wait()
        pltpu.make_async_copy(bias_hbm.at[iidx_v.at[j]], ib_v.at[j],
                              sem).wait()

    lanes = lax.iota(jnp.int32, NL)
    gidx = lanes * PSTR

    for j in range(NCH):
        def gbody(g, carry, j=j):
            r0 = g * NL
            for l in range(NL):
                r = r0 + l
                s0 = (urows_v[j, r, pl.ds(0, NL)]
                      * irows_v[j, r, pl.ds(0, NL)])
                s1 = (urows_v[j, r, pl.ds(NL, NL)]
                      * irows_v[j, r, pl.ds(NL, NL)])
                for d in range(2 * NL, DIM, 2 * NL):
                    s0 = s0 + (urows_v[j, r, pl.ds(d, NL)]
                               * irows_v[j, r, pl.ds(d, NL)])
                    s1 = s1 + (urows_v[j, r, pl.ds(d + NL, NL)]
                               * irows_v[j, r, pl.ds(d + NL, NL)])
                pt_v[pl.ds(l * PSTR, NL)] = s0 + s1
            tot = ub_v[j, pl.ds(r0, NL)] + ib_v[j, pl.ds(r0, NL)]
            for k in range(NL):
                tot = tot + plsc.load_gather(pt_v, [gidx + k])
            out_v[pl.ds(j * ICH + r0, NL)] = tot
            return carry
        lax.fori_loop(0, ICH // NL, gbody, 0)

    pltpu.sync_copy(out_v, out_hbm.at[pl.ds(base, BPW)])


@jax.jit
def _mf_sc(uidx, iidx, emb, bias):
    mesh = plsc.VectorSubcoreMesh(core_axis_name="c", subcore_axis_name="s")
    kfn = functools.partial(
        pl.kernel,
        mesh=mesh,
        out_type=jax.ShapeDtypeStruct((BATCH,), jnp.float32),
        scratch_types=[
            pltpu.VMEM((NCH, ICH), jnp.int32),
            pltpu.VMEM((NCH, ICH), jnp.int32),
            pltpu.VMEM((NCH, ICH, DIM), jnp.float32),
            pltpu.VMEM((NCH, ICH, DIM), jnp.float32),
            pltpu.VMEM((NCH, ICH), jnp.float32),
            pltpu.VMEM((NCH, ICH), jnp.float32),
            pltpu.VMEM((NL * PSTR,), jnp.float32),
            pltpu.VMEM((BPW,), jnp.float32),
            pltpu.SemaphoreType.DMA,
        ],
        compiler_params=pltpu.CompilerParams(needs_layout_passes=False,
                                             use_tc_tiling_on_sc=False),
    )(_sc_body)
    return kfn(uidx, iidx, emb, bias)


def kernel(user_idx, item_idx, user_emb, item_emb, user_bias, item_bias):
    nu = user_emb.shape[0]
    uidx = user_idx.astype(jnp.int32).reshape(NW, NCH, ICH)
    iidx = (item_idx.astype(jnp.int32) + nu).reshape(NW, NCH, ICH)
    emb = jnp.concatenate([user_emb, item_emb], axis=0)
    bias = jnp.concatenate([user_bias.reshape(-1), item_bias.reshape(-1)])
    return _mf_sc(uidx, iidx, emb, bias)


# speedup vs baseline: 1.5543x; 1.5543x over previous
import jax
import jax.numpy as jnp
from jax import lax
from jax.experimental import pallas as pl
from jax.experimental.pallas import tpu as pltpu
from jax.experimental.pallas import tpu_sc as plsc
import functools

BATCH = 16384
DIM = 64
NC = 2
NS = 16
NW = NC * NS
BPW = BATCH // NW
ICH = 128
NCH = BPW // ICH
NL = 16
PSTR = NL + 1


def _sc_body(uidx_hbm, iidx_hbm, uemb_hbm, iemb_hbm, ubias_hbm, ibias_hbm,
             out_hbm, uidx_v, iidx_v, urows_v, irows_v, ub_v, ib_v, pt_v,
             out_v, sem):
    wid = lax.axis_index("s") * NC + lax.axis_index("c")
    base = wid * BPW

    pltpu.sync_copy(uidx_hbm.at[wid], uidx_v)
    pltpu.sync_copy(iidx_hbm.at[wid], iidx_v)

    for j in range(NCH):
        pltpu.async_copy(uemb_hbm.at[uidx_v.at[j]], urows_v.at[j], sem)
        pltpu.async_copy(iemb_hbm.at[iidx_v.at[j]], irows_v.at[j], sem)
        pltpu.async_copy(ubias_hbm.at[uidx_v.at[j]], ub_v.at[j], sem)
        pltpu.async_copy(ibias_hbm.at[iidx_v.at[j]], ib_v.at[j], sem)
    for j in range(NCH):
        pltpu.make_async_copy(uemb_hbm.at[uidx_v.at[j]], urows_v.at[j],
                              sem).wait()
        pltpu.make_async_copy(iemb_hbm.at[iidx_v.at[j]], irows_v.at[j],
                              sem).wait()
        pltpu.make_async_copy(ubias_hbm.at[uidx_v.at[j]], ub_v.at[j],
                              sem).wait()
        pltpu.make_async_copy(ibias_hbm.at[iidx_v.at[j]], ib_v.at[j],
                              sem).wait()

    lanes = lax.iota(jnp.int32, NL)
    gidx = lanes * PSTR

    for j in range(NCH):
        def gbody(g, carry, j=j):
            r0 = g * NL
            for l in range(NL):
                r = r0 + l
                s0 = (urows_v[j, r, pl.ds(0, NL)]
                      * irows_v[j, r, pl.ds(0, NL)])
                s1 = (urows_v[j, r, pl.ds(NL, NL)]
                      * irows_v[j, r, pl.ds(NL, NL)])
                for d in range(2 * NL, DIM, 2 * NL):
                    s0 = s0 + (urows_v[j, r, pl.ds(d, NL)]
                               * irows_v[j, r, pl.ds(d, NL)])
                    s1 = s1 + (urows_v[j, r, pl.ds(d + NL, NL)]
                               * irows_v[j, r, pl.ds(d + NL, NL)])
                pt_v[pl.ds(l * PSTR, NL)] = s0 + s1
            tot = ub_v[j, pl.ds(r0, NL)] + ib_v[j, pl.ds(r0, NL)]
            for k in range(NL):
                tot = tot + plsc.load_gather(pt_v, [gidx + k])
            out_v[pl.ds(j * ICH + r0, NL)] = tot
            return carry
        lax.fori_loop(0, ICH // NL, gbody, 0)

    pltpu.sync_copy(out_v, out_hbm.at[pl.ds(base, BPW)])


@jax.jit
def _mf_sc(uidx, iidx, uemb, iemb, ubias, ibias):
    mesh = plsc.VectorSubcoreMesh(core_axis_name="c", subcore_axis_name="s")
    kfn = functools.partial(
        pl.kernel,
        mesh=mesh,
        out_type=jax.ShapeDtypeStruct((BATCH,), jnp.float32),
        scratch_types=[
            pltpu.VMEM((NCH, ICH), jnp.int32),
            pltpu.VMEM((NCH, ICH), jnp.int32),
            pltpu.VMEM((NCH, ICH, DIM), jnp.float32),
            pltpu.VMEM((NCH, ICH, DIM), jnp.float32),
            pltpu.VMEM((NCH, ICH), jnp.float32),
            pltpu.VMEM((NCH, ICH), jnp.float32),
            pltpu.VMEM((NL * PSTR,), jnp.float32),
            pltpu.VMEM((BPW,), jnp.float32),
            pltpu.SemaphoreType.DMA,
        ],
        compiler_params=pltpu.CompilerParams(needs_layout_passes=False,
                                             use_tc_tiling_on_sc=False),
    )(_sc_body)
    return kfn(uidx, iidx, uemb, iemb, ubias, ibias)


def kernel(user_idx, item_idx, user_emb, item_emb, user_bias, item_bias):
    uidx = user_idx.astype(jnp.int32).reshape(NW, NCH, ICH)
    iidx = item_idx.astype(jnp.int32).reshape(NW, NCH, ICH)
    return _mf_sc(uidx, iidx, user_emb, item_emb,
                  user_bias.reshape(-1), item_bias.reshape(-1))
